# baseline (speedup 1.0000x reference)
import jax
import jax.numpy as jnp
from jax import lax
from jax.experimental import pallas as pl
from jax.experimental.pallas import tpu as pltpu
from jax.experimental.pallas import tpu_sc as plsc

NUM_ROWS = 16384 * 50
DIM = 64
NUM_WORKERS = 32
R_PER_W = NUM_ROWS // NUM_WORKERS
ROWS = 200
SLOTS = R_PER_W // ROWS
NBUF = 4
GROUPS = SLOTS // NBUF - 1


def _emb_body(table_hbm, idx_hbm, out_hbm, idx_v, rows_v, *sems):
    gsem = sems[:NBUF]
    wsem = sems[NBUF:]
    wid = lax.axis_index("s") * 2 + lax.axis_index("c")
    base = wid * R_PER_W
    pltpu.sync_copy(idx_hbm.at[pl.ds(base, R_PER_W)], idx_v)

    def gather(slot, b):
        pltpu.async_copy(table_hbm.at[idx_v.at[pl.ds(slot * ROWS, ROWS)]],
                         rows_v.at[b], gsem[b])

    def gather_wait(slot, b):
        pltpu.make_async_copy(table_hbm.at[idx_v.at[pl.ds(slot * ROWS, ROWS)]],
                              rows_v.at[b], gsem[b]).wait()

    def wb(slot, b):
        pltpu.async_copy(rows_v.at[b],
                         out_hbm.at[pl.ds(base + slot * ROWS, ROWS)], wsem[b])

    def wb_wait(slot, b):
        pltpu.make_async_copy(rows_v.at[b],
                              out_hbm.at[pl.ds(base + slot * ROWS, ROWS)],
                              wsem[b]).wait()

    for b in range(NBUF):
        gather(b, b)

    def group(g, carry):
        for b in range(NBUF):
            slot = g * NBUF + b
            gather_wait(slot, b)
            wb(slot, b)
            wb_wait(slot, b)
            gather(slot + NBUF, b)
        return carry

    lax.fori_loop(0, GROUPS, group, 0)

    for b in range(NBUF):
        slot = GROUPS * NBUF + b
        gather_wait(slot, b)
        wb(slot, b)
    for b in range(NBUF):
        wb_wait(GROUPS * NBUF + b, b)


@jax.jit
def kernel(token_ids, indexing):
    mesh = plsc.VectorSubcoreMesh(core_axis_name="c", subcore_axis_name="s")
    out = pl.kernel(
        _emb_body,
        out_type=jax.ShapeDtypeStruct((NUM_ROWS, DIM), jnp.float32),
        mesh=mesh,
        scratch_types=[
            pltpu.VMEM((R_PER_W,), jnp.int32),
            pltpu.VMEM((NBUF, ROWS, DIM), jnp.float32),
        ] + [pltpu.SemaphoreType.DMA] * (2 * NBUF),
        compiler_params=pltpu.CompilerParams(use_tc_tiling_on_sc=False),
    )(indexing, token_ids.reshape(NUM_ROWS))
    return out.reshape(token_ids.shape + (DIM,))
